# baseline (speedup 1.0000x reference)
import functools

import jax
import jax.numpy as jnp
from jax import lax
from jax.experimental import pallas as pl
from jax.experimental.pallas import tpu as pltpu
from jax.experimental.pallas import tpu_sc as plsc

_VOCAB = 1000000
_EMBED = 64
_MAXLEN = 64
_SEGN = 2
_BATCH = 4096
_SEQ = 64

_NC = 2
_NS = 16
_NW = _NC * _NS
_ROWS = _BATCH * _SEQ
_RPW = _ROWS // _NW
_C = 512
_NCHUNK = _RPW // _C
_G = 128
_NG = _C // _G


def _body(ids_hbm, seg_hbm, word_hbm, pos_hbm, segtab_hbm, out_hbm,
          idsw_v, segw_v, idxg_v, gbuf_v, obuf_v, pos_v, segtab_v, gsem):
  wid = lax.axis_index("c") * _NS + lax.axis_index("s")
  base = wid * _RPW

  r0 = pl.multiple_of(wid * (_RPW // _G), 8)
  pltpu.sync_copy(ids_hbm.at[pl.ds(r0, _RPW // _G)], idsw_v)
  pltpu.sync_copy(seg_hbm.at[pl.ds(r0, _RPW // _G)], segw_v)
  pltpu.sync_copy(pos_hbm, pos_v)
  pltpu.sync_copy(segtab_hbm, segtab_v)

  def chunk_body(c, _):
    for jr in range(_NG):
      def shift_body(tt, _, jr=jr):
        d = pl.ds(tt * 16, 16)
        idxg_v[jr, d] = lax.shift_right_logical(idsw_v[c * _NG + jr, d], 1)
        return _
      lax.fori_loop(0, _G // 16, shift_body, None)

    cps = []
    for j in range(_NG):
      cps.append(pltpu.async_copy(word_hbm.at[idxg_v.at[j]],
                                  gbuf_v.at[pl.ds(j * _G, _G)], gsem))
    for cp in cps:
      cp.wait()

    def grp_body(t, _):
      idrow = c * _NG + lax.div(t, 8)
      col0 = lax.rem(t, 8) * 16
      ids16 = idsw_v[idrow, pl.ds(col0, 16)]
      seg16 = segw_v[idrow, pl.ds(col0, 16)]
      sbase = lax.rem(t, 4) * 16
      for i in range(16):
        r = t * 16 + i
        p64 = lax.mul(lax.rem(ids16[i], 2), 64)
        g64 = lax.mul(seg16[i], 64)
        s = sbase + i
        srow = lax.div(s, 2)
        scol = lax.rem(s, 2) * 64
        q = t * 8 + (i // 2)
        ocol = (i % 2) * 64
        for jj in range(_EMBED // 16):
          o = jj * 16
          v = gbuf_v[r, pl.ds(p64 + o, 16)]
          a = pos_v[srow, pl.ds(scol + o, 16)]
          b2 = segtab_v[0, pl.ds(g64 + o, 16)]
          obuf_v[q, pl.ds(ocol + o, 16)] = v + a + b2
      return _
    lax.fori_loop(0, _C // 16, grp_body, None)

    ob = pl.multiple_of((base + c * _C) // 2, _C // 2)
    pltpu.sync_copy(obuf_v, out_hbm.at[pl.ds(ob, _C // 2)])
    return _

  lax.fori_loop(0, _NCHUNK, chunk_body, None)


@functools.partial(
    pl.kernel,
    out_type=jax.ShapeDtypeStruct((_ROWS // 2, 2 * _EMBED), jnp.float32),
    mesh=plsc.VectorSubcoreMesh(core_axis_name="c", subcore_axis_name="s"),
    scratch_types=[
        pltpu.VMEM((_RPW // _G, _G), jnp.int32),
        pltpu.VMEM((_RPW // _G, _G), jnp.int32),
        pltpu.VMEM((_NG, _G), jnp.int32),
        pltpu.VMEM((_C, 2 * _EMBED), jnp.float32),
        pltpu.VMEM((_C // 2, 2 * _EMBED), jnp.float32),
        pltpu.VMEM((_MAXLEN // 2, 2 * _EMBED), jnp.float32),
        pltpu.VMEM((1, 2 * _EMBED), jnp.float32),
        pltpu.SemaphoreType.DMA,
    ],
    compiler_params=pltpu.CompilerParams(use_tc_tiling_on_sc=True),
)
def _embed_sc(*refs):
  _body(*refs)


@jax.jit
def kernel(input_ids, seg_ids, word_embed, pos_embed, seg_embed):
  ids2d = input_ids.astype(jnp.int32).reshape(_ROWS // _G, _G)
  seg2d = seg_ids.astype(jnp.int32).reshape(_ROWS // _G, _G)
  word2 = word_embed.reshape(_VOCAB // 2, 2 * _EMBED)
  pos2 = pos_embed.reshape(_MAXLEN // 2, 2 * _EMBED)
  segtab2 = seg_embed.reshape(1, 2 * _EMBED)
  out = _embed_sc(ids2d, seg2d, word2, pos2, segtab2)
  return out.reshape(_BATCH, _SEQ, _EMBED)
